# Initial kernel scaffold; baseline (speedup 1.0000x reference)
import functools

import jax
import jax.numpy as jnp
from jax import lax
from jax.experimental import pallas as pl
from jax.experimental.pallas import tpu as pltpu
from jax.experimental.pallas import tpu_sc as plsc

_IN = 4096
_OUT = 4096
_B = 64
_NNZ = 167772

_NC = 2
_NS = 16
_NW = _NC * _NS
_K = 128
_NCHUNK = -(-_NNZ // (_NW * _K))
_E = _NCHUNK * _K
_ROWS_PER_TILE = _OUT // _NS


def _sc_body(xt_hbm, col_hbm, row_hbm, w_hbm, out_hbm,
             colv, rowv, wv, gbuf, acc, sem):
    c = lax.axis_index("c")
    s = lax.axis_index("s")
    wid = c * _NS + s

    pltpu.sync_copy(col_hbm.at[wid], colv)
    pltpu.sync_copy(row_hbm.at[wid], rowv)
    pltpu.sync_copy(w_hbm.at[wid], wv)

    zeros = jnp.zeros((16,), jnp.float32)

    def _zero_row(r, carry):
        for j in range(_B // 16):
            gbuf[r, pl.ds(16 * j, 16)] = zeros
        return carry

    lax.fori_loop(0, _K, _zero_row, 0)
    pltpu.sync_copy(gbuf, acc.at[pl.ds(s * _ROWS_PER_TILE, _K)])
    pltpu.sync_copy(gbuf, acc.at[pl.ds(s * _ROWS_PER_TILE + _K, _K)])
    plsc.subcore_barrier()

    def _chunk(i, carry):
        pltpu.async_copy(xt_hbm.at[colv.at[i]], gbuf, sem).wait()

        def _scale(e, inner):
            w = wv[i * _K + e]
            for j in range(_B // 16):
                sl = pl.ds(16 * j, 16)
                gbuf[e, sl] = gbuf[e, sl] * w
            return inner

        lax.fori_loop(0, _K, _scale, 0)

        pltpu.sync_copy(gbuf, acc.at[rowv.at[i]], add=True)
        return carry

    lax.fori_loop(0, _NCHUNK, _chunk, 0)
    plsc.subcore_barrier()

    base = s * _ROWS_PER_TILE
    pltpu.sync_copy(acc.at[pl.ds(base, _ROWS_PER_TILE)],
                    out_hbm.at[c, pl.ds(base, _ROWS_PER_TILE)])


@jax.jit
def _sparse_linear(xt, colp, rowp, wp):
    mesh = plsc.VectorSubcoreMesh(core_axis_name="c", subcore_axis_name="s")
    run = pl.kernel(
        _sc_body,
        out_type=jax.ShapeDtypeStruct((_NC, _OUT, _B), jnp.float32),
        mesh=mesh,
        scratch_types=[
            pltpu.VMEM((_NCHUNK, _K), jnp.int32),
            pltpu.VMEM((_NCHUNK, _K), jnp.int32),
            pltpu.VMEM((_NCHUNK * _K,), jnp.float32),
            pltpu.VMEM((_K, _B), jnp.float32),
            pltpu.VMEM_SHARED((_OUT, _B), jnp.float32),
            pltpu.SemaphoreType.DMA,
        ],
    )
    return run(xt, colp, rowp, wp)


def kernel(inputs, weights, row, col):
    x = inputs.reshape(-1, _IN)
    xt = x.T

    pad = _NW * _E - _NNZ
    colp = jnp.concatenate([col, jnp.zeros((pad,), jnp.int32)])
    rowp = jnp.concatenate([row, jnp.zeros((pad,), jnp.int32)])
    wp = jnp.concatenate([weights, jnp.zeros((pad,), jnp.float32)])
    colp = colp.reshape(_NW, _NCHUNK, _K)
    rowp = rowp.reshape(_NW, _NCHUNK, _K)
    wp = wp.reshape(_NW, _NCHUNK * _K)

    part = _sparse_linear(xt, colp, rowp, wp)
    y = (part[0] + part[1]).T
    return y.reshape(*inputs.shape[:-1], _OUT)


# speedup vs baseline: 7.4893x; 7.4893x over previous
import functools

import jax
import jax.numpy as jnp
from jax import lax
from jax.experimental import pallas as pl
from jax.experimental.pallas import tpu as pltpu
from jax.experimental.pallas import tpu_sc as plsc

_IN = 4096
_OUT = 4096
_B = 64
_NNZ = 167772

_NC = 2
_NS = 16
_NW = _NC * _NS
_K = 128
_NCHUNK = -(-_NNZ // (_NW * _K))
_E = _NCHUNK * _K
_ROWS_PER_TILE = _OUT // _NS


def _sc_body(xt_hbm, col_hbm, row_hbm, w_hbm, out_hbm,
             colv, rowv, wv, gbuf, acc, sem):
    c = lax.axis_index("c")
    s = lax.axis_index("s")
    wid = c * _NS + s

    pltpu.sync_copy(col_hbm.at[wid], colv)
    pltpu.sync_copy(row_hbm.at[wid], rowv)
    pltpu.sync_copy(w_hbm.at[wid], wv)

    zeros = jnp.zeros((16,), jnp.float32)

    def _zero_row(r, carry):
        for j in range(_B // 16):
            gbuf[r, pl.ds(16 * j, 16)] = zeros
        return carry

    lax.fori_loop(0, _K, _zero_row, 0)
    pltpu.sync_copy(gbuf, acc.at[pl.ds(s * _ROWS_PER_TILE, _K)])
    pltpu.sync_copy(gbuf, acc.at[pl.ds(s * _ROWS_PER_TILE + _K, _K)])
    plsc.subcore_barrier()

    def _chunk(i, carry):
        pltpu.async_copy(xt_hbm.at[colv.at[i]], gbuf, sem).wait()

        def _scale_group(g, inner):
            base = g * 16
            wvec = wv[pl.ds(i * _K + base, 16)]
            for l in range(16):
                w = wvec[l]
                for j in range(_B // 16):
                    sl = pl.ds(16 * j, 16)
                    gbuf[base + l, sl] = gbuf[base + l, sl] * w
            return inner

        lax.fori_loop(0, _K // 16, _scale_group, 0)

        pltpu.sync_copy(gbuf, acc.at[rowv.at[i]], add=True)
        return carry

    lax.fori_loop(0, _NCHUNK, _chunk, 0)
    plsc.subcore_barrier()

    base = s * _ROWS_PER_TILE
    pltpu.sync_copy(acc.at[pl.ds(base, _ROWS_PER_TILE)],
                    out_hbm.at[c, pl.ds(base, _ROWS_PER_TILE)])


@jax.jit
def _sparse_linear(xt, colp, rowp, wp):
    mesh = plsc.VectorSubcoreMesh(core_axis_name="c", subcore_axis_name="s")
    run = pl.kernel(
        _sc_body,
        out_type=jax.ShapeDtypeStruct((_NC, _OUT, _B), jnp.float32),
        mesh=mesh,
        compiler_params=pltpu.CompilerParams(use_tc_tiling_on_sc=False),
        scratch_types=[
            pltpu.VMEM((_NCHUNK, _K), jnp.int32),
            pltpu.VMEM((_NCHUNK, _K), jnp.int32),
            pltpu.VMEM((_NCHUNK * _K,), jnp.float32),
            pltpu.VMEM((_K, _B), jnp.float32),
            pltpu.VMEM_SHARED((_OUT, _B), jnp.float32),
            pltpu.SemaphoreType.DMA,
        ],
    )
    return run(xt, colp, rowp, wp)


def kernel(inputs, weights, row, col):
    x = inputs.reshape(-1, _IN)
    xt = x.T

    pad = _NW * _E - _NNZ
    colp = jnp.concatenate([col, jnp.zeros((pad,), jnp.int32)])
    rowp = jnp.concatenate([row, jnp.zeros((pad,), jnp.int32)])
    wp = jnp.concatenate([weights, jnp.zeros((pad,), jnp.float32)])
    colp = colp.reshape(_NW, _NCHUNK, _K)
    rowp = rowp.reshape(_NW, _NCHUNK, _K)
    wp = wp.reshape(_NW, _NCHUNK * _K)

    part = _sparse_linear(xt, colp, rowp, wp)
    y = (part[0] + part[1]).T
    return y.reshape(*inputs.shape[:-1], _OUT)
